# baseline (speedup 1.0000x reference)
import jax
import jax.numpy as jnp
from jax import lax
from jax.experimental import pallas as pl
from jax.experimental.pallas import tpu as pltpu
from jax.experimental.pallas import tpu_sc as plsc

N_FIELDS = 26
CARD = 100000
DIM = 16
B = 16384
H1 = 64

NC, NS, L = 2, 16, 16
NW = NC * NS
ROWS_PER_W = B // NW
CHUNK_ROWS = 64
N_CHUNKS = ROWS_PER_W // CHUNK_ROWS
CHUNK_IDX = CHUNK_ROWS * N_FIELDS
N_STREAM = CHUNK_IDX // 128
OFF_PERIOD = 208
OFF_LEN = OFF_PERIOD + 128


CONV_CB = 1024
CONV_FULL = 2539
CONV_TAIL = CARD * N_FIELDS - CONV_FULL * CONV_CB


CONV_BPT = 80


def _conv_body(tt_hbm, tail_hbm, w_hbm, in0, in1, out0, out1,
               sem_in, sem_out):
    wid = lax.axis_index("s") * NC + lax.axis_index("c")
    ins, outs = (in0, in1), (out0, out1)
    iota16x = lax.iota(jnp.int32, L) * DIM

    def blk_of(k):
        return jnp.minimum(wid * CONV_BPT + k, CONV_FULL - 1)

    def in_descr(k, buf):
        c0 = blk_of(k) * CONV_CB
        return pltpu.make_async_copy(
            tt_hbm.at[:, pl.ds(c0, CONV_CB)], buf, sem_in)

    def out_descr(k, buf):
        c0 = blk_of(k) * CONV_CB
        return pltpu.make_async_copy(
            buf, w_hbm.at[pl.ds(c0 * DIM, CONV_CB * DIM)], sem_out)

    pos_d = [iota16x + d for d in range(DIM)]

    def transpose(in_buf, out_buf):
        for dlo in (0, DIM // 2):
            def jbody(j2, _, dlo=dlo):
                for j8 in range(8):
                    sbase = j2 * 2048 + j8 * 256
                    for d in range(dlo, dlo + DIM // 2):
                        v = in_buf[d, pl.ds(j2 * 128 + j8 * L, L)]
                        plsc.store_scatter(out_buf, [pos_d[d] + sbase], v)
                return 0

            lax.fori_loop(0, CONV_CB // 128, jbody, 0)

    @pl.when(wid == NW - 1)
    def _tail():
        pltpu.sync_copy(tail_hbm, out0.at[pl.ds(0, CONV_TAIL * DIM)])
        pltpu.sync_copy(out0.at[pl.ds(0, CONV_TAIL * DIM)],
                        w_hbm.at[pl.ds(CONV_FULL * CONV_CB * DIM,
                                       CONV_TAIL * DIM)])

    in_descr(0, in0).start()
    in_descr(1, in1).start()
    for k in (0, 1):
        in_descr(k, ins[k]).wait()
        transpose(ins[k], outs[k])
        out_descr(k, outs[k]).start()
        in_descr(k + 2, ins[k]).start()

    def pair_body(k2, _):
        for b in range(2):
            k = k2 * 2 + b
            out_descr(k - 2, outs[b]).wait()
            in_descr(k, ins[b]).wait()
            transpose(ins[b], outs[b])
            out_descr(k, outs[b]).start()
            in_descr(k + 2, ins[b]).start()
        return 0

    lax.fori_loop(1, CONV_BPT // 2, pair_body, 0)

    for b in range(2):
        in_descr(CONV_BPT + b, ins[b]).wait()
        out_descr(CONV_BPT - 2 + b, outs[b]).wait()


def _convert_table(tt, tail_flat):
    mesh = plsc.VectorSubcoreMesh(core_axis_name="c", subcore_axis_name="s")
    f = pl.kernel(
        _conv_body,
        out_type=jax.ShapeDtypeStruct((CARD * N_FIELDS * DIM,), jnp.float32),
        mesh=mesh,
        scratch_types=[
            pltpu.VMEM((DIM, CONV_CB), jnp.float32),
            pltpu.VMEM((DIM, CONV_CB), jnp.float32),
            pltpu.VMEM((DIM * CONV_CB,), jnp.float32),
            pltpu.VMEM((DIM * CONV_CB,), jnp.float32),
            pltpu.SemaphoreType.DMA,
            pltpu.SemaphoreType.DMA,
        ],
        compiler_params=pltpu.CompilerParams(
            needs_layout_passes=False, use_tc_tiling_on_sc=True),
    )
    return f(tt, tail_flat)


def _sc_body(x_hbm, off_hbm, table_hbm, wlin_hbm, pooled_hbm, lin_hbm,
             x_v, off_v, idx_v, rows_v, lin_v, pooled_v, linout_v, sem):
    wid = lax.axis_index("s") * NC + lax.axis_index("c")
    iota = lax.iota(jnp.int32, L)
    iota26 = iota * N_FIELDS
    zeros_i = jnp.zeros((L,), jnp.int32)

    pltpu.sync_copy(off_hbm, off_v)

    def chunk_body(ci, _):
        row0 = wid * ROWS_PER_W + ci * CHUNK_ROWS
        flat0 = row0 * N_FIELDS

        pltpu.sync_copy(x_hbm.at[pl.ds(flat0, CHUNK_IDX)], x_v)

        def idx_body(c, pstart):
            for v in range(8):
                xv = x_v[pl.ds(c * 128 + v * 16, L)]
                ov = off_v[pl.ds(pstart + v * 16, L)]
                idx_v[c, pl.ds(v * 16, L)] = xv + ov
            pnext = pstart + 128
            return jnp.where(pnext >= OFF_PERIOD, pnext - OFF_PERIOD, pnext)

        lax.fori_loop(0, N_STREAM, idx_body, jnp.int32(0))

        copies = []
        for c in range(N_STREAM):
            copies.append(pltpu.async_copy(
                table_hbm.at[idx_v.at[c]],
                rows_v.at[pl.ds(c * 128, 128)], sem))
        for c in range(N_STREAM):
            copies.append(pltpu.async_copy(
                wlin_hbm.at[idx_v.at[c]],
                lin_v.at[pl.ds(c * 128, 128)], sem))
        for cp in copies:
            cp.wait()

        def fm_body(r, _):
            p0 = r * N_FIELDS
            v = rows_v[p0]
            s = v
            q = v * v
            for f in range(1, N_FIELDS):
                v = rows_v[p0 + f]
                s = s + v
                q = q + v * v
            pooled_v[r] = s * s - q
            return 0

        lax.fori_loop(0, CHUNK_ROWS, fm_body, 0)

        for g in range(CHUNK_ROWS // L):
            pos0 = g * L * N_FIELDS
            acc = None
            for f in range(N_FIELDS):
                v = plsc.load_gather(lin_v, [iota26 + (pos0 + f)])
                acc = v if acc is None else acc + v
            linout_v[pl.ds(g * L, L)] = acc

        pltpu.sync_copy(pooled_v, pooled_hbm.at[pl.ds(row0, CHUNK_ROWS)])
        pltpu.sync_copy(linout_v, lin_hbm.at[pl.ds(row0, CHUNK_ROWS)])
        return 0

    lax.fori_loop(0, N_CHUNKS, chunk_body, 0)


def _sc_forward(x_flat, off_ext, table, w_lin):
    mesh = plsc.VectorSubcoreMesh(core_axis_name="c", subcore_axis_name="s")
    f = pl.kernel(
        _sc_body,
        out_type=(
            jax.ShapeDtypeStruct((B, DIM), jnp.float32),
            jax.ShapeDtypeStruct((B,), jnp.float32),
        ),
        mesh=mesh,
        scratch_types=[
            pltpu.VMEM((CHUNK_IDX,), jnp.int32),
            pltpu.VMEM((OFF_LEN,), jnp.int32),
            pltpu.VMEM((N_STREAM, 128), jnp.int32),
            pltpu.VMEM((CHUNK_IDX, DIM), jnp.float32),
            pltpu.VMEM((CHUNK_IDX,), jnp.float32),
            pltpu.VMEM((CHUNK_ROWS, DIM), jnp.float32),
            pltpu.VMEM((CHUNK_ROWS,), jnp.float32),
            pltpu.SemaphoreType.DMA,
        ],
        compiler_params=pltpu.CompilerParams(
            needs_layout_passes=False, use_tc_tiling_on_sc=False),
    )
    return f(x_flat, off_ext, table, w_lin)


def _mlp_body(pooled_ref, lin_ref, bias_ref, w1_ref, b1_ref, w2_ref, b2_ref,
              out_ref):
    h = jnp.dot(pooled_ref[...], w1_ref[...],
                preferred_element_type=jnp.float32) + b1_ref[...]
    h = jnp.maximum(h, 0.0)
    out = jnp.dot(h, w2_ref[...], preferred_element_type=jnp.float32)
    out_ref[...] = out + b2_ref[...] + bias_ref[...] + lin_ref[...]


def _mlp(pooled, lin, bias, W1, b1, W2, b2):
    return pl.pallas_call(
        _mlp_body,
        out_shape=jax.ShapeDtypeStruct((B, 1), jnp.float32),
    )(pooled, lin, bias.reshape(1, 1), W1, b1.reshape(1, H1), W2,
      b2.reshape(1, 1))


def kernel(x, table, w_lin, bias, W1, b1, W2, b2):
    x_flat = x.reshape(-1)
    off_ext = (jnp.arange(OFF_LEN, dtype=jnp.int32) % N_FIELDS) * CARD
    tail_flat = table[CONV_FULL * CONV_CB:].reshape(-1)
    w_flat = _convert_table(table.T, tail_flat)
    table_lin = w_flat.reshape(CARD * N_FIELDS, DIM)
    pooled, lin = _sc_forward(x_flat, off_ext, table_lin, w_lin.reshape(-1))
    return _mlp(pooled, lin.reshape(B, 1), bias, W1, b1, W2, b2)


# speedup vs baseline: 1.0578x; 1.0578x over previous
import jax
import jax.numpy as jnp
from jax import lax
from jax.experimental import pallas as pl
from jax.experimental.pallas import tpu as pltpu
from jax.experimental.pallas import tpu_sc as plsc

N_FIELDS = 26
CARD = 100000
DIM = 16
B = 16384
H1 = 64

NC, NS, L = 2, 16, 16
NW = NC * NS
ROWS_PER_W = B // NW
CHUNK_ROWS = 64
N_CHUNKS = ROWS_PER_W // CHUNK_ROWS
CHUNK_IDX = CHUNK_ROWS * N_FIELDS
N_STREAM = CHUNK_IDX // 128
OFF_PERIOD = 208
OFF_LEN = OFF_PERIOD + 128


CONV_CB = 1024
CONV_FULL = 2539
CONV_TAIL = CARD * N_FIELDS - CONV_FULL * CONV_CB


CONV_BPT = 80


def _conv_body(tt_hbm, tail_hbm, w_hbm, in0, in1, out0, out1,
               sem_in, sem_out):
    wid = lax.axis_index("s") * NC + lax.axis_index("c")
    ins, outs = (in0, in1), (out0, out1)
    iota16x = lax.iota(jnp.int32, L) * DIM

    def blk_of(k):
        return jnp.minimum(wid * CONV_BPT + k, CONV_FULL - 1)

    def in_descr(k, buf):
        c0 = blk_of(k) * CONV_CB
        return pltpu.make_async_copy(
            tt_hbm.at[:, pl.ds(c0, CONV_CB)], buf, sem_in)

    def out_descr(k, buf):
        c0 = blk_of(k) * CONV_CB
        return pltpu.make_async_copy(
            buf, w_hbm.at[pl.ds(c0 * DIM, CONV_CB * DIM)], sem_out)

    pos_d = [iota16x + d for d in range(DIM)]

    def transpose(in_buf, out_buf):
        for dlo in (0, DIM // 2):
            def jbody(j2, _, dlo=dlo):
                for j8 in range(8):
                    sbase = j2 * 2048 + j8 * 256
                    for d in range(dlo, dlo + DIM // 2):
                        v = in_buf[d, pl.ds(j2 * 128 + j8 * L, L)]
                        plsc.store_scatter(out_buf, [pos_d[d] + sbase], v)
                return 0

            lax.fori_loop(0, CONV_CB // 128, jbody, 0)

    @pl.when(wid == NW - 1)
    def _tail():
        pltpu.sync_copy(tail_hbm, out0.at[pl.ds(0, CONV_TAIL * DIM)])
        pltpu.sync_copy(out0.at[pl.ds(0, CONV_TAIL * DIM)],
                        w_hbm.at[pl.ds(CONV_FULL * CONV_CB * DIM,
                                       CONV_TAIL * DIM)])

    in_descr(0, in0).start()
    in_descr(1, in1).start()
    for k in (0, 1):
        in_descr(k, ins[k]).wait()
        transpose(ins[k], outs[k])
        out_descr(k, outs[k]).start()
        in_descr(k + 2, ins[k]).start()

    def pair_body(k2, _):
        for b in range(2):
            k = k2 * 2 + b
            out_descr(k - 2, outs[b]).wait()
            in_descr(k, ins[b]).wait()
            transpose(ins[b], outs[b])
            out_descr(k, outs[b]).start()
            in_descr(k + 2, ins[b]).start()
        return 0

    lax.fori_loop(1, CONV_BPT // 2, pair_body, 0)

    for b in range(2):
        in_descr(CONV_BPT + b, ins[b]).wait()
        out_descr(CONV_BPT - 2 + b, outs[b]).wait()


def _convert_table(tt, tail_flat):
    mesh = plsc.VectorSubcoreMesh(core_axis_name="c", subcore_axis_name="s")
    f = pl.kernel(
        _conv_body,
        out_type=jax.ShapeDtypeStruct((CARD * N_FIELDS * DIM,), jnp.float32),
        mesh=mesh,
        scratch_types=[
            pltpu.VMEM((DIM, CONV_CB), jnp.float32),
            pltpu.VMEM((DIM, CONV_CB), jnp.float32),
            pltpu.VMEM((DIM * CONV_CB,), jnp.float32),
            pltpu.VMEM((DIM * CONV_CB,), jnp.float32),
            pltpu.SemaphoreType.DMA,
            pltpu.SemaphoreType.DMA,
        ],
        compiler_params=pltpu.CompilerParams(
            needs_layout_passes=False, use_tc_tiling_on_sc=True),
    )
    return f(tt, tail_flat)


def _sc_body(x_hbm, off_hbm, table_hbm, wlin_hbm, pooled_hbm, lin_hbm,
             x_v, off_v, idx0, idx1, rows0, rows1, lin0, lin1,
             pooled_v, linout_v, sem0, sem1):
    wid = lax.axis_index("s") * NC + lax.axis_index("c")
    iota = lax.iota(jnp.int32, L)
    iota26 = iota * N_FIELDS
    bufs = ((idx0, rows0, lin0, sem0), (idx1, rows1, lin1, sem1))

    pltpu.sync_copy(off_hbm, off_v)

    def gather_descrs(buf):
        idx_v, rows_v, lin_v, sem = buf
        ds = []
        for c in range(N_STREAM):
            ds.append(pltpu.make_async_copy(
                table_hbm.at[idx_v.at[c]],
                rows_v.at[pl.ds(c * 128, 128)], sem))
        for c in range(N_STREAM):
            ds.append(pltpu.make_async_copy(
                wlin_hbm.at[idx_v.at[c]],
                lin_v.at[pl.ds(c * 128, 128)], sem))
        return ds

    def stage(ci, buf):
        idx_v = buf[0]
        flat0 = (wid * ROWS_PER_W + ci * CHUNK_ROWS) * N_FIELDS
        pltpu.sync_copy(x_hbm.at[pl.ds(flat0, CHUNK_IDX)], x_v)

        def idx_body(c, pstart):
            for v in range(8):
                xv = x_v[pl.ds(c * 128 + v * 16, L)]
                ov = off_v[pl.ds(pstart + v * 16, L)]
                idx_v[c, pl.ds(v * 16, L)] = xv + ov
            pnext = pstart + 128
            return jnp.where(pnext >= OFF_PERIOD, pnext - OFF_PERIOD, pnext)

        lax.fori_loop(0, N_STREAM, idx_body, jnp.int32(0))
        for d in gather_descrs(buf):
            d.start()

    def compute(ci, buf):
        _, rows_v, lin_v, _ = buf
        row0 = wid * ROWS_PER_W + ci * CHUNK_ROWS
        for d in gather_descrs(buf):
            d.wait()

        def fm_body(r, _):
            p0 = r * N_FIELDS
            v = rows_v[p0]
            s = v
            q = v * v
            for f in range(1, N_FIELDS):
                v = rows_v[p0 + f]
                s = s + v
                q = q + v * v
            pooled_v[r] = s * s - q
            return 0

        lax.fori_loop(0, CHUNK_ROWS, fm_body, 0)

        for g in range(CHUNK_ROWS // L):
            pos0 = g * L * N_FIELDS
            acc = None
            for f in range(N_FIELDS):
                v = plsc.load_gather(lin_v, [iota26 + (pos0 + f)])
                acc = v if acc is None else acc + v
            linout_v[pl.ds(g * L, L)] = acc

        pltpu.sync_copy(pooled_v, pooled_hbm.at[pl.ds(row0, CHUNK_ROWS)])
        pltpu.sync_copy(linout_v, lin_hbm.at[pl.ds(row0, CHUNK_ROWS)])

    stage(0, bufs[0])

    def pair_body(k2, _):
        k = k2 * 2
        stage(k + 1, bufs[1])
        compute(k, bufs[0])
        stage(k + 2, bufs[0])
        compute(k + 1, bufs[1])
        return 0

    lax.fori_loop(0, N_CHUNKS // 2 - 1, pair_body, 0)
    stage(N_CHUNKS - 1, bufs[1])
    compute(N_CHUNKS - 2, bufs[0])
    compute(N_CHUNKS - 1, bufs[1])


def _sc_forward(x_flat, off_ext, table, w_lin):
    mesh = plsc.VectorSubcoreMesh(core_axis_name="c", subcore_axis_name="s")
    f = pl.kernel(
        _sc_body,
        out_type=(
            jax.ShapeDtypeStruct((B, DIM), jnp.float32),
            jax.ShapeDtypeStruct((B,), jnp.float32),
        ),
        mesh=mesh,
        scratch_types=[
            pltpu.VMEM((CHUNK_IDX,), jnp.int32),
            pltpu.VMEM((OFF_LEN,), jnp.int32),
            pltpu.VMEM((N_STREAM, 128), jnp.int32),
            pltpu.VMEM((N_STREAM, 128), jnp.int32),
            pltpu.VMEM((CHUNK_IDX, DIM), jnp.float32),
            pltpu.VMEM((CHUNK_IDX, DIM), jnp.float32),
            pltpu.VMEM((CHUNK_IDX,), jnp.float32),
            pltpu.VMEM((CHUNK_IDX,), jnp.float32),
            pltpu.VMEM((CHUNK_ROWS, DIM), jnp.float32),
            pltpu.VMEM((CHUNK_ROWS,), jnp.float32),
            pltpu.SemaphoreType.DMA,
            pltpu.SemaphoreType.DMA,
        ],
        compiler_params=pltpu.CompilerParams(
            needs_layout_passes=False, use_tc_tiling_on_sc=False),
    )
    return f(x_flat, off_ext, table, w_lin)


def _mlp_body(pooled_ref, lin_ref, bias_ref, w1_ref, b1_ref, w2_ref, b2_ref,
              out_ref):
    h = jnp.dot(pooled_ref[...], w1_ref[...],
                preferred_element_type=jnp.float32) + b1_ref[...]
    h = jnp.maximum(h, 0.0)
    out = jnp.dot(h, w2_ref[...], preferred_element_type=jnp.float32)
    out_ref[...] = out + b2_ref[...] + bias_ref[...] + lin_ref[...]


def _mlp(pooled, lin, bias, W1, b1, W2, b2):
    return pl.pallas_call(
        _mlp_body,
        out_shape=jax.ShapeDtypeStruct((B, 1), jnp.float32),
    )(pooled, lin, bias.reshape(1, 1), W1, b1.reshape(1, H1), W2,
      b2.reshape(1, 1))


def kernel(x, table, w_lin, bias, W1, b1, W2, b2):
    x_flat = x.reshape(-1)
    off_ext = (jnp.arange(OFF_LEN, dtype=jnp.int32) % N_FIELDS) * CARD
    tail_flat = table[CONV_FULL * CONV_CB:].reshape(-1)
    w_flat = _convert_table(table.T, tail_flat)
    table_lin = w_flat.reshape(CARD * N_FIELDS, DIM)
    pooled, lin = _sc_forward(x_flat, off_ext, table_lin, w_lin.reshape(-1))
    return _mlp(pooled, lin.reshape(B, 1), bias, W1, b1, W2, b2)


# speedup vs baseline: 1.0582x; 1.0004x over previous
import jax
import jax.numpy as jnp
from jax import lax
from jax.experimental import pallas as pl
from jax.experimental.pallas import tpu as pltpu
from jax.experimental.pallas import tpu_sc as plsc

N_FIELDS = 26
CARD = 100000
DIM = 16
B = 16384
H1 = 64

NC, NS, L = 2, 16, 16
NW = NC * NS
ROWS_PER_W = B // NW
CHUNK_ROWS = 64
N_CHUNKS = ROWS_PER_W // CHUNK_ROWS
CHUNK_IDX = CHUNK_ROWS * N_FIELDS
N_STREAM = CHUNK_IDX // 128
OFF_PERIOD = 208
OFF_LEN = OFF_PERIOD + 128


CONV_CB = 512
CONV_FULL = 5078
CONV_TAIL = CARD * N_FIELDS - CONV_FULL * CONV_CB


CONV_BPT = 160


def _conv_body(tt_hbm, tail_hbm, w_hbm, in0, in1, out0, out1,
               sem_in, sem_out):
    wid = lax.axis_index("s") * NC + lax.axis_index("c")
    ins, outs = (in0, in1), (out0, out1)
    iota16x = lax.iota(jnp.int32, L) * DIM

    def blk_of(k):
        return jnp.minimum(wid * CONV_BPT + k, CONV_FULL - 1)

    def in_descr(k, buf):
        c0 = blk_of(k) * CONV_CB
        return pltpu.make_async_copy(
            tt_hbm.at[:, pl.ds(c0, CONV_CB)], buf, sem_in)

    def out_descr(k, buf):
        c0 = blk_of(k) * CONV_CB
        return pltpu.make_async_copy(
            buf, w_hbm.at[pl.ds(c0 * DIM, CONV_CB * DIM)], sem_out)

    pos_d = [iota16x + d for d in range(DIM)]

    def transpose(in_buf, out_buf):
        for dlo in (0, DIM // 2):
            def jbody(j2, _, dlo=dlo):
                for j8 in range(8):
                    sbase = j2 * 2048 + j8 * 256
                    for d in range(dlo, dlo + DIM // 2):
                        v = in_buf[d, pl.ds(j2 * 128 + j8 * L, L)]
                        plsc.store_scatter(out_buf, [pos_d[d] + sbase], v)
                return 0

            lax.fori_loop(0, CONV_CB // 128, jbody, 0)

    @pl.when(wid == NW - 1)
    def _tail():
        pltpu.sync_copy(tail_hbm, out0.at[pl.ds(0, CONV_TAIL * DIM)])
        pltpu.sync_copy(out0.at[pl.ds(0, CONV_TAIL * DIM)],
                        w_hbm.at[pl.ds(CONV_FULL * CONV_CB * DIM,
                                       CONV_TAIL * DIM)])

    in_descr(0, in0).start()
    in_descr(1, in1).start()
    for k in (0, 1):
        in_descr(k, ins[k]).wait()
        transpose(ins[k], outs[k])
        out_descr(k, outs[k]).start()
        in_descr(k + 2, ins[k]).start()

    def pair_body(k2, _):
        for b in range(2):
            k = k2 * 2 + b
            out_descr(k - 2, outs[b]).wait()
            in_descr(k, ins[b]).wait()
            transpose(ins[b], outs[b])
            out_descr(k, outs[b]).start()
            in_descr(k + 2, ins[b]).start()
        return 0

    lax.fori_loop(1, CONV_BPT // 2, pair_body, 0)

    for b in range(2):
        in_descr(CONV_BPT + b, ins[b]).wait()
        out_descr(CONV_BPT - 2 + b, outs[b]).wait()


def _convert_table(tt, tail_flat):
    mesh = plsc.VectorSubcoreMesh(core_axis_name="c", subcore_axis_name="s")
    f = pl.kernel(
        _conv_body,
        out_type=jax.ShapeDtypeStruct((CARD * N_FIELDS * DIM,), jnp.float32),
        mesh=mesh,
        scratch_types=[
            pltpu.VMEM((DIM, CONV_CB), jnp.float32),
            pltpu.VMEM((DIM, CONV_CB), jnp.float32),
            pltpu.VMEM((DIM * CONV_CB,), jnp.float32),
            pltpu.VMEM((DIM * CONV_CB,), jnp.float32),
            pltpu.SemaphoreType.DMA,
            pltpu.SemaphoreType.DMA,
        ],
        compiler_params=pltpu.CompilerParams(
            needs_layout_passes=False, use_tc_tiling_on_sc=True),
    )
    return f(tt, tail_flat)


def _sc_body(x_hbm, off_hbm, table_hbm, wlin_hbm, pooled_hbm, lin_hbm,
             x_v, off_v, idx0, idx1, rows0, rows1, lin0, lin1,
             pooled_v, linout_v, sem0, sem1):
    wid = lax.axis_index("s") * NC + lax.axis_index("c")
    iota = lax.iota(jnp.int32, L)
    iota26 = iota * N_FIELDS
    bufs = ((idx0, rows0, lin0, sem0), (idx1, rows1, lin1, sem1))

    pltpu.sync_copy(off_hbm, off_v)

    def gather_descrs(buf):
        idx_v, rows_v, lin_v, sem = buf
        ds = []
        for c in range(N_STREAM):
            ds.append(pltpu.make_async_copy(
                table_hbm.at[idx_v.at[c]],
                rows_v.at[pl.ds(c * 128, 128)], sem))
        for c in range(N_STREAM):
            ds.append(pltpu.make_async_copy(
                wlin_hbm.at[idx_v.at[c]],
                lin_v.at[pl.ds(c * 128, 128)], sem))
        return ds

    def stage(ci, buf):
        idx_v = buf[0]
        flat0 = (wid * ROWS_PER_W + ci * CHUNK_ROWS) * N_FIELDS
        pltpu.sync_copy(x_hbm.at[pl.ds(flat0, CHUNK_IDX)], x_v)

        def idx_body(c, pstart):
            for v in range(8):
                xv = x_v[pl.ds(c * 128 + v * 16, L)]
                ov = off_v[pl.ds(pstart + v * 16, L)]
                idx_v[c, pl.ds(v * 16, L)] = xv + ov
            pnext = pstart + 128
            return jnp.where(pnext >= OFF_PERIOD, pnext - OFF_PERIOD, pnext)

        lax.fori_loop(0, N_STREAM, idx_body, jnp.int32(0))
        for d in gather_descrs(buf):
            d.start()

    def compute(ci, buf):
        _, rows_v, lin_v, _ = buf
        row0 = wid * ROWS_PER_W + ci * CHUNK_ROWS
        for d in gather_descrs(buf):
            d.wait()

        def fm_body(r, _):
            p0 = r * N_FIELDS
            v = rows_v[p0]
            s = v
            q = v * v
            for f in range(1, N_FIELDS):
                v = rows_v[p0 + f]
                s = s + v
                q = q + v * v
            pooled_v[r] = s * s - q
            return 0

        lax.fori_loop(0, CHUNK_ROWS, fm_body, 0)

        for g in range(CHUNK_ROWS // L):
            pos0 = g * L * N_FIELDS
            acc = None
            for f in range(N_FIELDS):
                v = plsc.load_gather(lin_v, [iota26 + (pos0 + f)])
                acc = v if acc is None else acc + v
            linout_v[pl.ds(g * L, L)] = acc

        pltpu.sync_copy(pooled_v, pooled_hbm.at[pl.ds(row0, CHUNK_ROWS)])
        pltpu.sync_copy(linout_v, lin_hbm.at[pl.ds(row0, CHUNK_ROWS)])

    stage(0, bufs[0])

    def pair_body(k2, _):
        k = k2 * 2
        stage(k + 1, bufs[1])
        compute(k, bufs[0])
        stage(k + 2, bufs[0])
        compute(k + 1, bufs[1])
        return 0

    lax.fori_loop(0, N_CHUNKS // 2 - 1, pair_body, 0)
    stage(N_CHUNKS - 1, bufs[1])
    compute(N_CHUNKS - 2, bufs[0])
    compute(N_CHUNKS - 1, bufs[1])


def _sc_forward(x_flat, off_ext, table, w_lin):
    mesh = plsc.VectorSubcoreMesh(core_axis_name="c", subcore_axis_name="s")
    f = pl.kernel(
        _sc_body,
        out_type=(
            jax.ShapeDtypeStruct((B, DIM), jnp.float32),
            jax.ShapeDtypeStruct((B,), jnp.float32),
        ),
        mesh=mesh,
        scratch_types=[
            pltpu.VMEM((CHUNK_IDX,), jnp.int32),
            pltpu.VMEM((OFF_LEN,), jnp.int32),
            pltpu.VMEM((N_STREAM, 128), jnp.int32),
            pltpu.VMEM((N_STREAM, 128), jnp.int32),
            pltpu.VMEM((CHUNK_IDX, DIM), jnp.float32),
            pltpu.VMEM((CHUNK_IDX, DIM), jnp.float32),
            pltpu.VMEM((CHUNK_IDX,), jnp.float32),
            pltpu.VMEM((CHUNK_IDX,), jnp.float32),
            pltpu.VMEM((CHUNK_ROWS, DIM), jnp.float32),
            pltpu.VMEM((CHUNK_ROWS,), jnp.float32),
            pltpu.SemaphoreType.DMA,
            pltpu.SemaphoreType.DMA,
        ],
        compiler_params=pltpu.CompilerParams(
            needs_layout_passes=False, use_tc_tiling_on_sc=False),
    )
    return f(x_flat, off_ext, table, w_lin)


def _mlp_body(pooled_ref, lin_ref, bias_ref, w1_ref, b1_ref, w2_ref, b2_ref,
              out_ref):
    h = jnp.dot(pooled_ref[...], w1_ref[...],
                preferred_element_type=jnp.float32) + b1_ref[...]
    h = jnp.maximum(h, 0.0)
    out = jnp.dot(h, w2_ref[...], preferred_element_type=jnp.float32)
    out_ref[...] = out + b2_ref[...] + bias_ref[...] + lin_ref[...]


def _mlp(pooled, lin, bias, W1, b1, W2, b2):
    return pl.pallas_call(
        _mlp_body,
        out_shape=jax.ShapeDtypeStruct((B, 1), jnp.float32),
    )(pooled, lin, bias.reshape(1, 1), W1, b1.reshape(1, H1), W2,
      b2.reshape(1, 1))


def kernel(x, table, w_lin, bias, W1, b1, W2, b2):
    x_flat = x.reshape(-1)
    off_ext = (jnp.arange(OFF_LEN, dtype=jnp.int32) % N_FIELDS) * CARD
    tail_flat = table[CONV_FULL * CONV_CB:].reshape(-1)
    w_flat = _convert_table(table.T, tail_flat)
    table_lin = w_flat.reshape(CARD * N_FIELDS, DIM)
    pooled, lin = _sc_forward(x_flat, off_ext, table_lin, w_lin.reshape(-1))
    return _mlp(pooled, lin.reshape(B, 1), bias, W1, b1, W2, b2)
